# baseline (speedup 1.0000x reference)
import jax
import jax.numpy as jnp
from jax import lax
from jax.experimental import pallas as pl
from jax.experimental.pallas import tpu as pltpu
from jax.experimental.pallas import tpu_sc as plsc

_SHIFT = 0.6931471805599453

NC = 2
NS = 16
NW = NC * NS
CH = 64
GRAIN = NW * CH * 2


def _softplus(x):
    return jnp.maximum(x, 0.0) + jnp.log(1.0 + jnp.exp(-jnp.abs(x)))


def _xw_body(nf_ref, w1_ref, x_ref):
    x_ref[...] = jnp.dot(nf_ref[...], w1_ref[...],
                         preferred_element_type=jnp.float32)


def _filter_body(ef_ref, wf1_ref, bf1_ref, wf2_ref, bf2_ref, out_ref):
    h = jnp.dot(ef_ref[...].astype(jnp.bfloat16),
                wf1_ref[...].astype(jnp.bfloat16),
                preferred_element_type=jnp.float32)
    h = _softplus(h + bf1_ref[...]) - _SHIFT
    h = jnp.dot(h.astype(jnp.bfloat16), wf2_ref[...].astype(jnp.bfloat16),
                preferred_element_type=jnp.float32)
    out_ref[...] = _softplus(h + bf2_ref[...]) - _SHIFT


def _final_body(s0_ref, s1_ref, ct_ref, nf_ref,
                w2_ref, b2_ref, w3_ref, b3_ref, out_ref):
    cnt = jnp.sum(ct_ref[...], axis=1, keepdims=True)
    mean = (s0_ref[...] + s1_ref[...]) / jnp.maximum(cnt, 1.0)
    h = jnp.dot(mean, w2_ref[...], preferred_element_type=jnp.float32)
    h = _softplus(h + b2_ref[...]) - _SHIFT
    h = jnp.dot(h, w3_ref[...], preferred_element_type=jnp.float32)
    out_ref[...] = nf_ref[...] + h + b3_ref[...]


def _make_sc_kernel(dim, nch, np_rows, zr):
    mesh = plsc.VectorSubcoreMesh(core_axis_name="c", subcore_axis_name="s",
                                  num_cores=NC, num_subcores=NS)

    def body(x_hbm, idxc_hbm, wf_hbm, zrow_hbm, sums_hbm,
             idx_a, idx_b, w_a, w_b, rows_a, rows_b, sums_sh,
             sga, sgb, swa, swb, ssa, ssb):
        cid = lax.axis_index("c")
        sid = lax.axis_index("s")
        wid = cid * NS + sid
        g0 = wid * nch

        r0 = sid * zr
        pltpu.sync_copy(zrow_hbm, sums_sh.at[pl.ds(r0, zr), :])
        plsc.subcore_barrier()

        def prefetch(ci, idx_v, w_v, rows_v, sg, sw):
            gi = g0 + ci
            pltpu.sync_copy(idxc_hbm.at[gi], idx_v)
            pltpu.async_copy(wf_hbm.at[pl.ds(gi * CH, CH), :], w_v, sw)
            pltpu.async_copy(x_hbm.at[idx_v.at[0]], rows_v, sg)

        def process(ci, idx_v, w_v, rows_v, sg, sw, ss):
            gi = g0 + ci
            pltpu.make_async_copy(wf_hbm.at[pl.ds(gi * CH, CH), :], w_v,
                                  sw).wait()
            pltpu.make_async_copy(x_hbm.at[idx_v.at[0]], rows_v, sg).wait()

            def mul_row(r, c2):
                for k in range(dim // 16):
                    s = pl.ds(k * 16, 16)
                    rows_v[r, s] = rows_v[r, s] * w_v[r, s]
                return c2
            lax.fori_loop(0, CH, mul_row, 0)
            pltpu.async_copy(rows_v, sums_sh.at[idx_v.at[1]], ss, add=True)

        def wait_scatter(idx_v, rows_v, ss):
            pltpu.make_async_copy(rows_v, sums_sh.at[idx_v.at[1]], ss).wait()

        ng = nch // 2
        prefetch(0, idx_a, w_a, rows_a, sga, swa)

        def pair_body(g, carry):
            @pl.when(g > 0)
            def _():
                wait_scatter(idx_b, rows_b, ssb)
            prefetch(2 * g + 1, idx_b, w_b, rows_b, sgb, swb)
            process(2 * g, idx_a, w_a, rows_a, sga, swa, ssa)
            wait_scatter(idx_a, rows_a, ssa)

            @pl.when(g < ng - 1)
            def _():
                prefetch(2 * g + 2, idx_a, w_a, rows_a, sga, swa)
            process(2 * g + 1, idx_b, w_b, rows_b, sgb, swb, ssb)
            return carry
        lax.fori_loop(0, ng, pair_body, 0)
        wait_scatter(idx_b, rows_b, ssb)

        plsc.subcore_barrier()
        ob = cid * np_rows + r0
        pltpu.sync_copy(sums_sh.at[pl.ds(r0, zr), :],
                        sums_hbm.at[pl.ds(ob, zr), :])

    return pl.kernel(
        body,
        out_type=jax.ShapeDtypeStruct((NC * np_rows, dim), jnp.float32),
        mesh=mesh,
        scratch_types=[
            pltpu.VMEM((2, CH), jnp.int32),
            pltpu.VMEM((2, CH), jnp.int32),
            pltpu.VMEM((CH, dim), jnp.float32),
            pltpu.VMEM((CH, dim), jnp.float32),
            pltpu.VMEM((CH, dim), jnp.float32),
            pltpu.VMEM((CH, dim), jnp.float32),
            pltpu.VMEM_SHARED((np_rows, dim), jnp.float32),
            pltpu.SemaphoreType.DMA,
            pltpu.SemaphoreType.DMA,
            pltpu.SemaphoreType.DMA,
            pltpu.SemaphoreType.DMA,
            pltpu.SemaphoreType.DMA,
            pltpu.SemaphoreType.DMA,
        ],
    )


def _make_cnt_kernel(ept, np_rows):
    mesh = plsc.VectorSubcoreMesh(core_axis_name="c", subcore_axis_name="s",
                                  num_cores=NC, num_subcores=NS)

    def body(idx1_hbm, cnts_hbm, idx1_v, hist_v):
        cid = lax.axis_index("c")
        sid = lax.axis_index("s")
        wid = cid * NS + sid

        def zero_body(i, carry):
            hist_v[pl.ds(i * 16, 16)] = jnp.zeros((16,), jnp.float32)
            return carry
        lax.fori_loop(0, np_rows // 16, zero_body, 0)

        pltpu.sync_copy(idx1_hbm.at[pl.ds(wid * ept, ept)], idx1_v)
        ones16 = jnp.ones((16,), jnp.float32)

        def chunk_body(t, carry):
            for j in range(4):
                iv = idx1_v[pl.ds((t * 4 + j) * 16, 16)]
                plsc.addupdate_scatter(hist_v, [iv], ones16)
            return carry
        lax.fori_loop(0, ept // 64, chunk_body, 0)

        pltpu.sync_copy(hist_v, cnts_hbm.at[wid])

    return pl.kernel(
        body,
        out_type=jax.ShapeDtypeStruct((NW, np_rows), jnp.float32),
        mesh=mesh,
        compiler_params=pltpu.CompilerParams(needs_layout_passes=False),
        scratch_types=[
            pltpu.VMEM((ept,), jnp.int32),
            pltpu.VMEM((np_rows,), jnp.float32),
        ],
    )


def kernel(node_fea, idx1, idx2, edge_fea, W1, Wf1, bf1, Wf2, bf2, W2, b2,
           W3, b3):
    n, d = node_fea.shape
    e, de = edge_fea.shape

    e_pad = -(-e // GRAIN) * GRAIN
    pad = e_pad - e
    ept = e_pad // NW
    nch = ept // CH
    idx1p = jnp.concatenate([idx1.astype(jnp.int32),
                             jnp.full((pad,), n, jnp.int32)])
    idx2p = jnp.concatenate([idx2.astype(jnp.int32),
                             jnp.zeros((pad,), jnp.int32)])
    efp = jnp.concatenate([edge_fea, jnp.zeros((pad, de), edge_fea.dtype)])
    idxc = jnp.stack([idx2p.reshape(-1, CH), idx1p.reshape(-1, CH)], axis=1)

    np_rows = (n + NS * 8) // (NS * 8) * (NS * 8)
    zr = np_rows // NS

    x = pl.pallas_call(
        _xw_body,
        out_shape=jax.ShapeDtypeStruct((n, d), jnp.float32),
    )(node_fea, W1)

    be = 1024
    wf = pl.pallas_call(
        _filter_body,
        grid=(e_pad // be,),
        in_specs=[
            pl.BlockSpec((be, de), lambda i: (i, 0)),
            pl.BlockSpec((de, d), lambda i: (0, 0)),
            pl.BlockSpec((1, d), lambda i: (0, 0)),
            pl.BlockSpec((d, d), lambda i: (0, 0)),
            pl.BlockSpec((1, d), lambda i: (0, 0)),
        ],
        out_specs=pl.BlockSpec((be, d), lambda i: (i, 0)),
        out_shape=jax.ShapeDtypeStruct((e_pad, d), jnp.float32),
    )(efp, Wf1, bf1.reshape(1, d), Wf2, bf2.reshape(1, d))

    zrow = jnp.zeros((zr, d), jnp.float32)
    sums_p = _make_sc_kernel(d, nch, np_rows, zr)(x, idxc, wf, zrow)

    cnts_p = _make_cnt_kernel(ept, np_rows)(idx1p)

    s0 = sums_p[:n]
    s1 = sums_p[np_rows:np_rows + n]
    ct = jnp.transpose(cnts_p)[:n]

    bn = 1000
    out = pl.pallas_call(
        _final_body,
        grid=(n // bn,),
        in_specs=[
            pl.BlockSpec((bn, d), lambda i: (i, 0)),
            pl.BlockSpec((bn, d), lambda i: (i, 0)),
            pl.BlockSpec((bn, NW), lambda i: (i, 0)),
            pl.BlockSpec((bn, d), lambda i: (i, 0)),
            pl.BlockSpec((d, d), lambda i: (0, 0)),
            pl.BlockSpec((1, d), lambda i: (0, 0)),
            pl.BlockSpec((d, d), lambda i: (0, 0)),
            pl.BlockSpec((1, d), lambda i: (0, 0)),
        ],
        out_specs=pl.BlockSpec((bn, d), lambda i: (i, 0)),
        out_shape=jax.ShapeDtypeStruct((n, d), jnp.float32),
    )(s0, s1, ct, node_fea, W2, b2.reshape(1, d), W3, b3.reshape(1, d))
    return out


# speedup vs baseline: 1.0367x; 1.0367x over previous
import jax
import jax.numpy as jnp
from jax import lax
from jax.experimental import pallas as pl
from jax.experimental.pallas import tpu as pltpu
from jax.experimental.pallas import tpu_sc as plsc

_SHIFT = 0.6931471805599453

NC = 2
NS = 16
NW = NC * NS
CH = 64


def _softplus(x):
    return jnp.maximum(x, 0.0) + jnp.log(1.0 + jnp.exp(-jnp.abs(x)))


def _xw_body(nf_ref, w1_ref, x_ref):
    x_ref[...] = jnp.dot(nf_ref[...], w1_ref[...],
                         preferred_element_type=jnp.float32)


def _filter_body(ef_ref, wf1_ref, bf1_ref, wf2_ref, bf2_ref, out_ref):
    h = jnp.dot(ef_ref[...], wf1_ref[...], preferred_element_type=jnp.float32)
    h = _softplus(h + bf1_ref[...]) - _SHIFT
    h = jnp.dot(h, wf2_ref[...], preferred_element_type=jnp.float32)
    out_ref[...] = _softplus(h + bf2_ref[...]) - _SHIFT


def _final_body(s0_ref, s1_ref, ct_ref, nf_ref,
                w2_ref, b2_ref, w3_ref, b3_ref, out_ref):
    cnt = jnp.sum(ct_ref[...], axis=1, keepdims=True)
    mean = (s0_ref[...] + s1_ref[...]) / jnp.maximum(cnt, 1.0)
    h = jnp.dot(mean, w2_ref[...], preferred_element_type=jnp.float32)
    h = _softplus(h + b2_ref[...]) - _SHIFT
    h = jnp.dot(h, w3_ref[...], preferred_element_type=jnp.float32)
    out_ref[...] = nf_ref[...] + h + b3_ref[...]


def _make_sc_kernel(dim, nch, np_rows, zr):
    mesh = plsc.VectorSubcoreMesh(core_axis_name="c", subcore_axis_name="s",
                                  num_cores=NC, num_subcores=NS)

    nch_a, nch_b = nch

    def body(x_hbm, idxc_hbm, wf_hbm, zrow_hbm, sums_hbm,
             idx_a, idx_b, w_a, w_b, rows_a, rows_b, sums_sh,
             sga, sgb, swa, swb, ssa, ssb):
        cid = lax.axis_index("c")
        sid = lax.axis_index("s")
        g0 = jnp.where(cid == 0, sid * nch_a, NS * nch_a + sid * nch_b)
        ntile = jnp.where(cid == 0, nch_a, nch_b)

        r0 = sid * zr
        pltpu.sync_copy(zrow_hbm, sums_sh.at[pl.ds(r0, zr), :])
        plsc.subcore_barrier()

        def prefetch(ci, idx_v, w_v, rows_v, sg, sw):
            gi = g0 + ci
            pltpu.sync_copy(idxc_hbm.at[gi], idx_v)
            pltpu.async_copy(wf_hbm.at[pl.ds(gi * CH, CH), :], w_v, sw)
            pltpu.async_copy(x_hbm.at[idx_v.at[0]], rows_v, sg)

        def process(ci, idx_v, w_v, rows_v, sg, sw, ss):
            gi = g0 + ci
            pltpu.make_async_copy(wf_hbm.at[pl.ds(gi * CH, CH), :], w_v,
                                  sw).wait()
            pltpu.make_async_copy(x_hbm.at[idx_v.at[0]], rows_v, sg).wait()

            def mul_row(r, c2):
                for k in range(dim // 16):
                    s = pl.ds(k * 16, 16)
                    rows_v[r, s] = rows_v[r, s] * w_v[r, s]
                return c2
            lax.fori_loop(0, CH, mul_row, 0)
            pltpu.async_copy(rows_v, sums_sh.at[idx_v.at[1]], ss, add=True)

        def wait_scatter(idx_v, rows_v, ss):
            pltpu.make_async_copy(rows_v, sums_sh.at[idx_v.at[1]], ss).wait()

        ng = ntile // 2
        prefetch(0, idx_a, w_a, rows_a, sga, swa)

        def pair_body(g, carry):
            @pl.when(g > 0)
            def _():
                wait_scatter(idx_b, rows_b, ssb)
            prefetch(2 * g + 1, idx_b, w_b, rows_b, sgb, swb)
            process(2 * g, idx_a, w_a, rows_a, sga, swa, ssa)
            wait_scatter(idx_a, rows_a, ssa)

            @pl.when(g < ng - 1)
            def _():
                prefetch(2 * g + 2, idx_a, w_a, rows_a, sga, swa)
            process(2 * g + 1, idx_b, w_b, rows_b, sgb, swb, ssb)
            return carry
        lax.fori_loop(0, ng, pair_body, 0)
        wait_scatter(idx_b, rows_b, ssb)

        plsc.subcore_barrier()
        ob = cid * np_rows + r0
        pltpu.sync_copy(sums_sh.at[pl.ds(r0, zr), :],
                        sums_hbm.at[pl.ds(ob, zr), :])

    return pl.kernel(
        body,
        out_type=jax.ShapeDtypeStruct((NC * np_rows, dim), jnp.float32),
        mesh=mesh,
        scratch_types=[
            pltpu.VMEM((2, CH), jnp.int32),
            pltpu.VMEM((2, CH), jnp.int32),
            pltpu.VMEM((CH, dim), jnp.float32),
            pltpu.VMEM((CH, dim), jnp.float32),
            pltpu.VMEM((CH, dim), jnp.float32),
            pltpu.VMEM((CH, dim), jnp.float32),
            pltpu.VMEM_SHARED((np_rows, dim), jnp.float32),
            pltpu.SemaphoreType.DMA,
            pltpu.SemaphoreType.DMA,
            pltpu.SemaphoreType.DMA,
            pltpu.SemaphoreType.DMA,
            pltpu.SemaphoreType.DMA,
            pltpu.SemaphoreType.DMA,
        ],
    )


def _make_cnt_kernel(ept, np_rows):
    mesh = plsc.VectorSubcoreMesh(core_axis_name="c", subcore_axis_name="s",
                                  num_cores=NC, num_subcores=NS)

    def body(idx1_hbm, cnts_hbm, idx1_v, hist_v):
        cid = lax.axis_index("c")
        sid = lax.axis_index("s")
        wid = cid * NS + sid

        def zero_body(i, carry):
            hist_v[pl.ds(i * 16, 16)] = jnp.zeros((16,), jnp.float32)
            return carry
        lax.fori_loop(0, np_rows // 16, zero_body, 0)

        pltpu.sync_copy(idx1_hbm.at[pl.ds(wid * ept, ept)], idx1_v)
        ones16 = jnp.ones((16,), jnp.float32)

        def chunk_body(t, carry):
            for j in range(4):
                iv = idx1_v[pl.ds((t * 4 + j) * 16, 16)]
                plsc.addupdate_scatter(hist_v, [iv], ones16)
            return carry
        lax.fori_loop(0, ept // 64, chunk_body, 0)

        pltpu.sync_copy(hist_v, cnts_hbm.at[wid])

    return pl.kernel(
        body,
        out_type=jax.ShapeDtypeStruct((NW, np_rows), jnp.float32),
        mesh=mesh,
        compiler_params=pltpu.CompilerParams(needs_layout_passes=False),
        scratch_types=[
            pltpu.VMEM((ept,), jnp.int32),
            pltpu.VMEM((np_rows,), jnp.float32),
        ],
    )


def kernel(node_fea, idx1, idx2, edge_fea, W1, Wf1, bf1, Wf2, bf2, W2, b2,
           W3, b3):
    n, d = node_fea.shape
    e, de = edge_fea.shape

    s_tot = -(-e // (NS * CH))
    nch_a = max(2, 2 * round(s_tot * 0.6 / 2))
    nch_b = max(2, -(-(s_tot - nch_a) // 2) * 2)
    e_pad = NS * CH * (nch_a + nch_b)
    pad = e_pad - e
    ept = e_pad // NW
    idx1p = jnp.concatenate([idx1.astype(jnp.int32),
                             jnp.full((pad,), n, jnp.int32)])
    idx2p = jnp.concatenate([idx2.astype(jnp.int32),
                             jnp.zeros((pad,), jnp.int32)])
    idxc = jnp.stack([idx2p.reshape(-1, CH), idx1p.reshape(-1, CH)], axis=1)

    np_rows = (n + NS * 8) // (NS * 8) * (NS * 8)
    zr = np_rows // NS

    x = pl.pallas_call(
        _xw_body,
        out_shape=jax.ShapeDtypeStruct((n, d), jnp.float32),
    )(node_fea, W1)

    be = next(b for b in (1024, 640, 512, 320, 256, 128, 64, 32, 16, 8)
              if e % b == 0)
    wf = pl.pallas_call(
        _filter_body,
        grid=(e // be,),
        in_specs=[
            pl.BlockSpec((be, de), lambda i: (i, 0)),
            pl.BlockSpec((de, d), lambda i: (0, 0)),
            pl.BlockSpec((1, d), lambda i: (0, 0)),
            pl.BlockSpec((d, d), lambda i: (0, 0)),
            pl.BlockSpec((1, d), lambda i: (0, 0)),
        ],
        out_specs=pl.BlockSpec((be, d), lambda i: (i, 0)),
        out_shape=jax.ShapeDtypeStruct((e_pad, d), jnp.float32),
    )(edge_fea, Wf1, bf1.reshape(1, d), Wf2, bf2.reshape(1, d))

    zrow = jnp.zeros((zr, d), jnp.float32)
    sums_p = _make_sc_kernel(d, (nch_a, nch_b), np_rows, zr)(x, idxc, wf, zrow)

    cnts_p = _make_cnt_kernel(ept, np_rows)(idx1p)

    s0 = sums_p[:n]
    s1 = sums_p[np_rows:np_rows + n]
    ct = jnp.transpose(cnts_p)[:n]

    bn = 1000
    out = pl.pallas_call(
        _final_body,
        grid=(n // bn,),
        in_specs=[
            pl.BlockSpec((bn, d), lambda i: (i, 0)),
            pl.BlockSpec((bn, d), lambda i: (i, 0)),
            pl.BlockSpec((bn, NW), lambda i: (i, 0)),
            pl.BlockSpec((bn, d), lambda i: (i, 0)),
            pl.BlockSpec((d, d), lambda i: (0, 0)),
            pl.BlockSpec((1, d), lambda i: (0, 0)),
            pl.BlockSpec((d, d), lambda i: (0, 0)),
            pl.BlockSpec((1, d), lambda i: (0, 0)),
        ],
        out_specs=pl.BlockSpec((bn, d), lambda i: (i, 0)),
        out_shape=jax.ShapeDtypeStruct((n, d), jnp.float32),
    )(s0, s1, ct, node_fea, W2, b2.reshape(1, d), W3, b3.reshape(1, d))
    return out


# speedup vs baseline: 1.3605x; 1.3124x over previous
import jax
import jax.numpy as jnp
from jax import lax
from jax.experimental import pallas as pl
from jax.experimental.pallas import tpu as pltpu
from jax.experimental.pallas import tpu_sc as plsc

_SHIFT = 0.6931471805599453

NC = 2
NS = 16
NW = NC * NS
CH = 64


def _softplus(x):
    return jnp.maximum(x, 0.0) + jnp.log(1.0 + jnp.exp(-jnp.abs(x)))


def _xw_body(nf_ref, w1_ref, x_ref):
    x_ref[...] = jnp.dot(nf_ref[...], w1_ref[...],
                         preferred_element_type=jnp.float32)


def _filter_body(ef_ref, wf1_ref, bf1_ref, wf2_ref, bf2_ref, out_ref):
    h = jnp.dot(ef_ref[...], wf1_ref[...], preferred_element_type=jnp.float32)
    h = _softplus(h + bf1_ref[...]) - _SHIFT
    h = jnp.dot(h, wf2_ref[...], preferred_element_type=jnp.float32)
    out_ref[...] = _softplus(h + bf2_ref[...]) - _SHIFT


def _final_body(s0_ref, s1_ref, ct_ref, nf_ref,
                w2_ref, b2_ref, w3_ref, b3_ref, out_ref):
    cnt = jnp.sum(ct_ref[...], axis=1, keepdims=True)
    mean = (s0_ref[...] + s1_ref[...]) / jnp.maximum(cnt, 1.0)
    h = jnp.dot(mean, w2_ref[...], preferred_element_type=jnp.float32)
    h = _softplus(h + b2_ref[...]) - _SHIFT
    h = jnp.dot(h, w3_ref[...], preferred_element_type=jnp.float32)
    out_ref[...] = nf_ref[...] + h + b3_ref[...]


def _make_sc_kernel(dim, nch, np_rows, zr):
    mesh = plsc.VectorSubcoreMesh(core_axis_name="c", subcore_axis_name="s",
                                  num_cores=NC, num_subcores=NS)

    nch_a, nch_b = nch

    def body(x_hbm, idxc_hbm, wf_hbm, zrow_hbm, sums_hbm,
             idx_a, idx_b, w_a, w_b, rows_a, rows_b, sums_sh,
             sga, sgb, swa, swb, ssa, ssb):
        cid = lax.axis_index("c")
        sid = lax.axis_index("s")
        g0 = jnp.where(cid == 0, sid * nch_a, NS * nch_a + sid * nch_b)
        ntile = jnp.where(cid == 0, nch_a, nch_b)

        r0 = sid * zr
        pltpu.sync_copy(zrow_hbm, sums_sh.at[pl.ds(r0, zr), :])
        plsc.subcore_barrier()

        def prefetch(ci, idx_v, w_v, rows_v, sg, sw):
            gi = g0 + ci
            pltpu.sync_copy(idxc_hbm.at[gi], idx_v)
            pltpu.async_copy(wf_hbm.at[pl.ds(gi * CH, CH), :], w_v, sw)
            pltpu.async_copy(x_hbm.at[idx_v.at[0]], rows_v, sg)

        def process(ci, idx_v, w_v, rows_v, sg, sw, ss):
            gi = g0 + ci
            pltpu.make_async_copy(wf_hbm.at[pl.ds(gi * CH, CH), :], w_v,
                                  sw).wait()
            pltpu.make_async_copy(x_hbm.at[idx_v.at[0]], rows_v, sg).wait()

            def mul_row(r, c2):
                for k in range(dim // 16):
                    s = pl.ds(k * 16, 16)
                    rows_v[r, s] = rows_v[r, s] * w_v[r, s]
                return c2
            lax.fori_loop(0, CH, mul_row, 0)
            pltpu.async_copy(rows_v, sums_sh.at[idx_v.at[1]], ss, add=True)

        def wait_scatter(idx_v, rows_v, ss):
            pltpu.make_async_copy(rows_v, sums_sh.at[idx_v.at[1]], ss).wait()

        ng = ntile // 2
        prefetch(0, idx_a, w_a, rows_a, sga, swa)

        def pair_body(g, carry):
            @pl.when(g > 0)
            def _():
                wait_scatter(idx_b, rows_b, ssb)
            prefetch(2 * g + 1, idx_b, w_b, rows_b, sgb, swb)
            process(2 * g, idx_a, w_a, rows_a, sga, swa, ssa)
            wait_scatter(idx_a, rows_a, ssa)

            @pl.when(g < ng - 1)
            def _():
                prefetch(2 * g + 2, idx_a, w_a, rows_a, sga, swa)
            process(2 * g + 1, idx_b, w_b, rows_b, sgb, swb, ssb)
            return carry
        lax.fori_loop(0, ng, pair_body, 0)
        wait_scatter(idx_b, rows_b, ssb)

        plsc.subcore_barrier()
        ob = cid * np_rows + r0
        pltpu.sync_copy(sums_sh.at[pl.ds(r0, zr), :],
                        sums_hbm.at[pl.ds(ob, zr), :])

    return pl.kernel(
        body,
        out_type=jax.ShapeDtypeStruct((NC * np_rows, dim), jnp.float32),
        mesh=mesh,
        scratch_types=[
            pltpu.VMEM((2, CH), jnp.int32),
            pltpu.VMEM((2, CH), jnp.int32),
            pltpu.VMEM((CH, dim), jnp.float32),
            pltpu.VMEM((CH, dim), jnp.float32),
            pltpu.VMEM((CH, dim), jnp.float32),
            pltpu.VMEM((CH, dim), jnp.float32),
            pltpu.VMEM_SHARED((np_rows, dim), jnp.float32),
            pltpu.SemaphoreType.DMA,
            pltpu.SemaphoreType.DMA,
            pltpu.SemaphoreType.DMA,
            pltpu.SemaphoreType.DMA,
            pltpu.SemaphoreType.DMA,
            pltpu.SemaphoreType.DMA,
        ],
    )


def _make_cnt_kernel(ept, np_rows):
    mesh = plsc.VectorSubcoreMesh(core_axis_name="c", subcore_axis_name="s",
                                  num_cores=NC, num_subcores=NS)

    def body(idx1_hbm, cnts_hbm, idx1_v, hist_v):
        cid = lax.axis_index("c")
        sid = lax.axis_index("s")
        wid = cid * NS + sid

        def zero_body(i, carry):
            hist_v[pl.ds(i * 16, 16)] = jnp.zeros((16,), jnp.float32)
            return carry
        lax.fori_loop(0, np_rows // 16, zero_body, 0)

        pltpu.sync_copy(idx1_hbm.at[pl.ds(wid * ept, ept)], idx1_v)
        ones16 = jnp.ones((16,), jnp.float32)

        def chunk_body(t, carry):
            for j in range(4):
                iv = idx1_v[pl.ds((t * 4 + j) * 16, 16)]
                plsc.addupdate_scatter(hist_v, [iv], ones16)
            return carry
        lax.fori_loop(0, ept // 64, chunk_body, 0)

        pltpu.sync_copy(hist_v, cnts_hbm.at[wid])

    return pl.kernel(
        body,
        out_type=jax.ShapeDtypeStruct((NW, np_rows), jnp.float32),
        mesh=mesh,
        compiler_params=pltpu.CompilerParams(needs_layout_passes=False),
        scratch_types=[
            pltpu.VMEM((ept,), jnp.int32),
            pltpu.VMEM((np_rows,), jnp.float32),
        ],
    )


def kernel(node_fea, idx1, idx2, edge_fea, W1, Wf1, bf1, Wf2, bf2, W2, b2,
           W3, b3):
    n, d = node_fea.shape
    e, de = edge_fea.shape

    s_tot = -(-e // (NS * CH))
    nch_a = max(2, 2 * round(s_tot * 0.565 / 2))
    nch_b = max(2, -(-(s_tot - nch_a) // 2) * 2)
    e_pad = NS * CH * (nch_a + nch_b)
    pad = e_pad - e
    ept = e_pad // NW
    idx1p = jnp.concatenate([idx1.astype(jnp.int32),
                             jnp.full((pad,), n, jnp.int32)])
    idx2p = jnp.concatenate([idx2.astype(jnp.int32),
                             jnp.zeros((pad,), jnp.int32)])
    idxc = jnp.stack([idx2p.reshape(-1, CH), idx1p.reshape(-1, CH)], axis=1)

    np_rows = (n + NS * 8) // (NS * 8) * (NS * 8)
    zr = np_rows // NS

    x = pl.pallas_call(
        _xw_body,
        out_shape=jax.ShapeDtypeStruct((n, d), jnp.float32),
    )(node_fea, W1)

    be = next(b for b in (2000, 1600, 1024, 640, 512, 320, 256, 128, 64, 32,
                          16, 8) if e % b == 0)
    wf = pl.pallas_call(
        _filter_body,
        grid=(e // be,),
        in_specs=[
            pl.BlockSpec((be, de), lambda i: (i, 0)),
            pl.BlockSpec((de, d), lambda i: (0, 0)),
            pl.BlockSpec((1, d), lambda i: (0, 0)),
            pl.BlockSpec((d, d), lambda i: (0, 0)),
            pl.BlockSpec((1, d), lambda i: (0, 0)),
        ],
        out_specs=pl.BlockSpec((be, d), lambda i: (i, 0)),
        out_shape=jax.ShapeDtypeStruct((e_pad, d), jnp.float32),
    )(edge_fea, Wf1, bf1.reshape(1, d), Wf2, bf2.reshape(1, d))

    zrow = jnp.zeros((zr, d), jnp.float32)
    sums_p = _make_sc_kernel(d, (nch_a, nch_b), np_rows, zr)(x, idxc, wf, zrow)

    cnts_p = _make_cnt_kernel(ept, np_rows)(idx1p)

    s0 = sums_p[:n]
    s1 = sums_p[np_rows:np_rows + n]
    ct = jnp.transpose(cnts_p)[:n]

    bn = 1000
    out = pl.pallas_call(
        _final_body,
        grid=(n // bn,),
        in_specs=[
            pl.BlockSpec((bn, d), lambda i: (i, 0)),
            pl.BlockSpec((bn, d), lambda i: (i, 0)),
            pl.BlockSpec((bn, NW), lambda i: (i, 0)),
            pl.BlockSpec((bn, d), lambda i: (i, 0)),
            pl.BlockSpec((d, d), lambda i: (0, 0)),
            pl.BlockSpec((1, d), lambda i: (0, 0)),
            pl.BlockSpec((d, d), lambda i: (0, 0)),
            pl.BlockSpec((1, d), lambda i: (0, 0)),
        ],
        out_specs=pl.BlockSpec((bn, d), lambda i: (i, 0)),
        out_shape=jax.ShapeDtypeStruct((n, d), jnp.float32),
    )(s0, s1, ct, node_fea, W2, b2.reshape(1, d), W3, b3.reshape(1, d))
    return out
